# baseline (speedup 1.0000x reference)
import functools
import math

import jax
import jax.numpy as jnp
from jax import lax
from jax.experimental import pallas as pl
from jax.experimental.pallas import tpu as pltpu
from jax.experimental.pallas import tpu_sc as plsc

BLK = 256
CB = 256
CH = 32
NW = 32


def _router_body(x_ref, wr_ref, br_ref, dst_ref, wgt_ref, cnt_ref, meta_ref,
                 *, S, E, H):
    nchunks = S // CB
    wr = wr_ref[...]
    br = br_ref[...]
    tri = (jax.lax.broadcasted_iota(jnp.int32, (CB, CB), 0)
           >= jax.lax.broadcasted_iota(jnp.int32, (CB, CB), 1)
           ).astype(jnp.float32)

    def pass1(i, carry):
        x = x_ref[pl.ds(i * CB, CB), :]
        logits = jnp.dot(x, wr, preferred_element_type=jnp.float32) + br
        m = jnp.max(logits, axis=-1, keepdims=True)
        p = jnp.exp(logits - m)
        p = p / jnp.sum(p, axis=-1, keepdims=True)

        lane = jax.lax.broadcasted_iota(jnp.int32, (CB, E), 1)
        m1 = jnp.max(p, axis=-1, keepdims=True)
        am1 = jnp.min(jnp.where(p == m1, lane, E), axis=-1, keepdims=True)
        p2 = jnp.where(lane == am1, -jnp.inf, p)
        m2 = jnp.max(p2, axis=-1, keepdims=True)
        am2 = jnp.min(jnp.where(p2 == m2, lane, E), axis=-1, keepdims=True)
        denom = m1 + m2
        w0 = m1 / denom
        w1 = m2 / denom

        lane8 = jax.lax.broadcasted_iota(jnp.int32, (CB, 8), 1)
        wgt_ref[pl.ds(i * CB, CB), :] = (
            jnp.where(lane8 == 0, w0, 0.0) + jnp.where(lane8 == 1, w1, 0.0))

        onehot = ((lane == am1) | (lane == am2)).astype(jnp.float32)
        csum = jnp.dot(tri, onehot, preferred_element_type=jnp.float32) + carry
        rank0 = jnp.sum(jnp.where(lane == am1, csum, 0.0), axis=-1,
                        keepdims=True) - 1.0
        rank1 = jnp.sum(jnp.where(lane == am2, csum, 0.0), axis=-1,
                        keepdims=True) - 1.0
        meta_ref[pl.ds(i * CB, CB), :] = (
            jnp.where(lane8 == 0, am1, 0)
            + jnp.where(lane8 == 1, am2, 0)
            + jnp.where(lane8 == 2, rank0.astype(jnp.int32), 0)
            + jnp.where(lane8 == 3, rank1.astype(jnp.int32), 0))
        return csum[CB - 1:CB, :]

    totals = lax.fori_loop(0, nchunks, pass1,
                           jnp.zeros((1, E), jnp.float32))

    lane_e = jax.lax.broadcasted_iota(jnp.int32, (1, E), 1)
    n_s = [jnp.sum(jnp.where(lane_e == e, totals, 0.0)) for e in range(E)]
    start_s = []
    run = jnp.zeros((), jnp.float32)
    for e in range(E):
        start_s.append(run)
        run = run + jnp.floor((n_s[e] + (BLK - 1)) / BLK) * BLK

    lane8r = jax.lax.broadcasted_iota(jnp.int32, (8, 8), 1)
    cnt = jnp.zeros((8, 8), jnp.float32)
    for e in range(E):
        cnt = cnt + jnp.where(lane8r == e, n_s[e], 0.0)
    cnt_ref[...] = cnt.astype(jnp.int32)

    def pass2(i, _):
        mc = meta_ref[pl.ds(i * CB, CB), :]
        am1 = mc[:, 0:1]
        am2 = mc[:, 1:2]
        r0 = mc[:, 2:3]
        r1 = mc[:, 3:4]
        s0 = jnp.zeros_like(r0)
        s1 = jnp.zeros_like(r1)
        for e in range(E):
            st = start_s[e].astype(jnp.int32)
            s0 = jnp.where(am1 == e, st, s0)
            s1 = jnp.where(am2 == e, st, s1)
        d0 = s0 + r0
        d1 = s1 + r1
        lane8 = jax.lax.broadcasted_iota(jnp.int32, (CB, 8), 1)
        dst_ref[pl.ds(i * CB, CB), :] = (
            jnp.where(lane8 == 0, d0, 0) + jnp.where(lane8 == 1, d1, 0))
        return 0

    lax.fori_loop(0, nchunks, pass2, 0)


def _router(x, Wr, br):
    S, H = x.shape
    E = Wr.shape[1]
    body = functools.partial(_router_body, S=S, E=E, H=H)
    return pl.pallas_call(
        body,
        out_shape=[
            jax.ShapeDtypeStruct((S, 8), jnp.int32),
            jax.ShapeDtypeStruct((S, 8), jnp.float32),
            jax.ShapeDtypeStruct((8, 8), jnp.int32),
        ],
        scratch_shapes=[pltpu.VMEM((S, 8), jnp.int32)],
    )(x, Wr, br.reshape(1, E))


def _dispatch_sc(x, d0, d1, R):
    S, H = x.shape
    nchunks = S // CH
    per_w = nchunks // NW
    d0c = d0.reshape(nchunks, CH)
    d1c = d1.reshape(nchunks, CH)
    mesh = plsc.VectorSubcoreMesh(core_axis_name="c", subcore_axis_name="s")

    @functools.partial(
        pl.kernel, mesh=mesh,
        out_type=jax.ShapeDtypeStruct((R, H), jnp.float32),
        scratch_types=[
            pltpu.VMEM((CH, H), jnp.float32),
            pltpu.VMEM((CH,), jnp.int32),
            pltpu.VMEM((CH,), jnp.int32),
            pltpu.SemaphoreType.DMA,
        ],
    )
    def k(x_hbm, d0_hbm, d1_hbm, xs_hbm, xbuf, i0, i1, sem):
        wid = lax.axis_index("s") * 2 + lax.axis_index("c")
        for j in range(per_w):
            c = wid * per_w + j
            pltpu.sync_copy(x_hbm.at[pl.ds(c * CH, CH)], xbuf)
            pltpu.sync_copy(d0_hbm.at[c], i0)
            pltpu.sync_copy(d1_hbm.at[c], i1)
            pltpu.async_copy(xbuf, xs_hbm.at[i0], sem).wait()
            pltpu.async_copy(xbuf, xs_hbm.at[i1], sem).wait()

    return k(x, d0c, d1c)


def _ffn_body(g_ref, a_ref, x_ref, lng_ref, lnb_ref, w1_ref, b1_ref, w2_ref,
              b2_ref, y_ref, acc_ref, w1buf, w2buf, w1sem, w2sem,
              *, ff_t, t_max, blk_ff, n_e):
    e = pl.program_id(0)
    ff = pl.program_id(1)
    t = pl.program_id(2)
    sl = pl.ds(t * BLK, BLK)

    p = e * ff_t + ff
    npass = n_e * ff_t
    slot = lax.rem(p, 2)

    def w1_copy(pp, s):
        ee = pp // ff_t
        fz = lax.rem(pp, ff_t)
        return pltpu.make_async_copy(
            w1_ref.at[ee, :, pl.ds(fz * blk_ff, blk_ff)], w1buf.at[s],
            w1sem.at[s])

    def w2_copy(pp, s):
        ee = pp // ff_t
        fz = lax.rem(pp, ff_t)
        return pltpu.make_async_copy(
            w2_ref.at[ee, pl.ds(fz * blk_ff, blk_ff), :], w2buf.at[s],
            w2sem.at[s])

    @pl.when(t == 0)
    def _prefetch():
        @pl.when(p == 0)
        def _first():
            w1_copy(p, slot).start()
            w2_copy(p, slot).start()

        @pl.when(p + 1 < npass)
        def _next():
            w1_copy(p + 1, 1 - slot).start()
            w2_copy(p + 1, 1 - slot).start()

        w1_copy(p, slot).wait()
        w2_copy(p, slot).wait()

    @pl.when(a_ref[e * t_max + t] > 0)
    def _():
        x = x_ref[...]
        mu = jnp.mean(x, axis=-1, keepdims=True)
        var = (jnp.mean(jnp.square(x), axis=-1, keepdims=True)
               - jnp.square(mu))
        xn = (x - mu) * jax.lax.rsqrt(var + 1e-5)
        xn = (xn * lng_ref[0, 0] + lnb_ref[0, 0]).astype(jnp.bfloat16)

        def ffn_with(w1v, w2v):
            h1 = jnp.dot(xn, w1v, preferred_element_type=jnp.float32)
            h1 = h1 + b1_ref[0, 0]
            h1 = 0.5 * h1 * (1.0 + jax.lax.erf(h1 * (1.0 / math.sqrt(2.0))))
            part = jnp.dot(h1.astype(jnp.bfloat16), w2v,
                           preferred_element_type=jnp.float32)

            @pl.when(ff == 0)
            def _init():
                acc_ref[sl, :] = part.astype(jnp.bfloat16)

            @pl.when(ff > 0)
            def _acc():
                acc_ref[sl, :] = (acc_ref[sl, :].astype(jnp.float32)
                                  + part).astype(jnp.bfloat16)

        @pl.when(slot == 0)
        def _even():
            ffn_with(w1buf[0], w2buf[0])

        @pl.when(slot == 1)
        def _odd():
            ffn_with(w1buf[1], w2buf[1])

        @pl.when(ff == ff_t - 1)
        def _flush():
            y_ref[...] = acc_ref[sl, :].astype(jnp.float32) + b2_ref[0, 0]


def _ffn_grouped(xs, g_clamp, active, ln_g, ln_b, W1, b1, W2, b2, t_max):
    R, H = xs.shape
    E, _, FF = W1.shape
    blk_ff = min(2048, FF)
    ff_t = FF // blk_ff
    W1 = W1.astype(jnp.bfloat16)
    W2 = W2.astype(jnp.bfloat16)
    body = functools.partial(_ffn_body, ff_t=ff_t, t_max=t_max,
                             blk_ff=blk_ff, n_e=E)

    def y_idx(e, ff, t, g, a):
        return (jnp.where(ff == ff_t - 1, g[e * t_max + t], g[e * t_max]), 0)

    grid_spec = pltpu.PrefetchScalarGridSpec(
        num_scalar_prefetch=2,
        grid=(E, ff_t, t_max),
        in_specs=[
            pl.BlockSpec((BLK, H), lambda e, ff, t, g, a: (g[e * t_max + t], 0)),
            pl.BlockSpec((1, 1, H), lambda e, ff, t, g, a: (e, 0, 0)),
            pl.BlockSpec((1, 1, H), lambda e, ff, t, g, a: (e, 0, 0)),
            pl.BlockSpec(memory_space=pl.ANY),
            pl.BlockSpec((1, 1, blk_ff),
                         lambda e, ff, t, g, a: (e * ff_t + ff, 0, 0)),
            pl.BlockSpec(memory_space=pl.ANY),
            pl.BlockSpec((1, 1, H), lambda e, ff, t, g, a: (e, 0, 0)),
        ],
        out_specs=pl.BlockSpec((BLK, H), y_idx),
        scratch_shapes=[
            pltpu.VMEM((t_max * BLK, H), jnp.bfloat16),
            pltpu.VMEM((2, H, blk_ff), jnp.bfloat16),
            pltpu.VMEM((2, blk_ff, H), jnp.bfloat16),
            pltpu.SemaphoreType.DMA((2,)),
            pltpu.SemaphoreType.DMA((2,)),
        ],
    )
    return pl.pallas_call(
        body,
        grid_spec=grid_spec,
        out_shape=jax.ShapeDtypeStruct((R, H), jnp.float32),
    )(g_clamp, active, xs, ln_g.reshape(E, 1, H), ln_b.reshape(E, 1, H),
      W1, b1.reshape(E * ff_t, 1, blk_ff), W2, b2.reshape(E, 1, H))


def _slot_gather_sc(y, d0, d1):
    R, H = y.shape
    S = d0.shape[0]
    nchunks = S // CH
    per_w = nchunks // NW
    d0c = d0.reshape(nchunks, CH)
    d1c = d1.reshape(nchunks, CH)
    mesh = plsc.VectorSubcoreMesh(core_axis_name="c", subcore_axis_name="s")

    @functools.partial(
        pl.kernel, mesh=mesh,
        out_type=[
            jax.ShapeDtypeStruct((S, H), jnp.float32),
            jax.ShapeDtypeStruct((S, H), jnp.float32),
        ],
        scratch_types=[
            pltpu.VMEM((CH, H), jnp.float32),
            pltpu.VMEM((CH,), jnp.int32),
            pltpu.SemaphoreType.DMA,
        ],
    )
    def k(y_hbm, d0_hbm, d1_hbm, y0_hbm, y1_hbm, buf, idx, sem):
        wid = lax.axis_index("s") * 2 + lax.axis_index("c")
        for j in range(per_w):
            c = wid * per_w + j
            pltpu.sync_copy(d0_hbm.at[c], idx)
            pltpu.async_copy(y_hbm.at[idx], buf, sem).wait()
            pltpu.sync_copy(buf, y0_hbm.at[pl.ds(c * CH, CH)])
            pltpu.sync_copy(d1_hbm.at[c], idx)
            pltpu.async_copy(y_hbm.at[idx], buf, sem).wait()
            pltpu.sync_copy(buf, y1_hbm.at[pl.ds(c * CH, CH)])

    return k(y, d0c, d1c)


def _combine_body(x_ref, y0_ref, y1_ref, w_ref, o_ref):
    w0 = w_ref[:, 0:1]
    w1 = w_ref[:, 1:2]
    o_ref[...] = x_ref[...] + w0 * y0_ref[...] + w1 * y1_ref[...]


def _combine(x, y0, y1, wgt):
    S, H = x.shape
    blk = 256
    grid = (S // blk,)
    return pl.pallas_call(
        _combine_body,
        grid=grid,
        in_specs=[
            pl.BlockSpec((blk, H), lambda s: (s, 0)),
            pl.BlockSpec((blk, H), lambda s: (s, 0)),
            pl.BlockSpec((blk, H), lambda s: (s, 0)),
            pl.BlockSpec((blk, 8), lambda s: (s, 0)),
        ],
        out_specs=pl.BlockSpec((blk, H), lambda s: (s, 0)),
        out_shape=jax.ShapeDtypeStruct((S, H), jnp.float32),
    )(x, y0, y1, wgt)


def kernel(hidden_states, Wr, br, ln_g, ln_b, W1, b1, W2, b2):
    B, S_in, H = hidden_states.shape
    S = B * S_in
    E = Wr.shape[1]
    CAP = 2
    x = hidden_states.reshape(S, H)

    dst, wgt, cnt = _router(x, Wr, br)
    d0 = dst[:, 0]
    d1 = dst[:, 1]

    t_max = S // BLK
    n = cnt[0, :E]
    tiles = (n + (BLK - 1)) // BLK
    excl = jnp.concatenate(
        [jnp.zeros((1,), jnp.int32), jnp.cumsum(tiles)[:-1].astype(jnp.int32)])
    t_iota = jnp.arange(t_max, dtype=jnp.int32)
    g_clamp = (excl[:, None]
               + jnp.minimum(t_iota[None, :],
                             jnp.maximum(tiles[:, None] - 1, 0))
               ).reshape(-1).astype(jnp.int32)
    active = (t_iota[None, :] < tiles[:, None]).reshape(-1).astype(jnp.int32)

    R = CAP * S + E * BLK
    xs = _dispatch_sc(x, d0, d1, R)
    y = _ffn_grouped(xs, g_clamp, active, ln_g, ln_b, W1, b1, W2, b2, t_max)
    y0, y1 = _slot_gather_sc(y, d0, d1)
    out = _combine(x, y0, y1, wgt)
    return out.reshape(B, S_in, H)


# speedup vs baseline: 1.0601x; 1.0601x over previous
import functools
import math

import jax
import jax.numpy as jnp
from jax import lax
from jax.experimental import pallas as pl
from jax.experimental.pallas import tpu as pltpu
from jax.experimental.pallas import tpu_sc as plsc

BLK = 256
CB = 256
CH = 32
NW = 32


def _router_body(x_ref, wr_ref, br_ref, dst_ref, wgt_ref, cnt_ref, meta_ref,
                 *, S, E, H):
    nchunks = S // CB
    wr = wr_ref[...]
    br = br_ref[...]
    tri = (jax.lax.broadcasted_iota(jnp.int32, (CB, CB), 0)
           >= jax.lax.broadcasted_iota(jnp.int32, (CB, CB), 1)
           ).astype(jnp.float32)

    def pass1(i, carry):
        x = x_ref[pl.ds(i * CB, CB), :]
        logits = jnp.dot(x, wr, preferred_element_type=jnp.float32) + br
        m = jnp.max(logits, axis=-1, keepdims=True)
        p = jnp.exp(logits - m)
        p = p / jnp.sum(p, axis=-1, keepdims=True)

        lane = jax.lax.broadcasted_iota(jnp.int32, (CB, E), 1)
        m1 = jnp.max(p, axis=-1, keepdims=True)
        am1 = jnp.min(jnp.where(p == m1, lane, E), axis=-1, keepdims=True)
        p2 = jnp.where(lane == am1, -jnp.inf, p)
        m2 = jnp.max(p2, axis=-1, keepdims=True)
        am2 = jnp.min(jnp.where(p2 == m2, lane, E), axis=-1, keepdims=True)
        denom = m1 + m2
        w0 = m1 / denom
        w1 = m2 / denom

        lane8 = jax.lax.broadcasted_iota(jnp.int32, (CB, 8), 1)
        wgt_ref[pl.ds(i * CB, CB), :] = (
            jnp.where(lane8 == 0, w0, 0.0) + jnp.where(lane8 == 1, w1, 0.0))

        onehot = ((lane == am1) | (lane == am2)).astype(jnp.float32)
        csum = jnp.dot(tri, onehot, preferred_element_type=jnp.float32) + carry
        rank0 = jnp.sum(jnp.where(lane == am1, csum, 0.0), axis=-1,
                        keepdims=True) - 1.0
        rank1 = jnp.sum(jnp.where(lane == am2, csum, 0.0), axis=-1,
                        keepdims=True) - 1.0
        meta_ref[pl.ds(i * CB, CB), :] = (
            jnp.where(lane8 == 0, am1, 0)
            + jnp.where(lane8 == 1, am2, 0)
            + jnp.where(lane8 == 2, rank0.astype(jnp.int32), 0)
            + jnp.where(lane8 == 3, rank1.astype(jnp.int32), 0))
        return csum[CB - 1:CB, :]

    totals = lax.fori_loop(0, nchunks, pass1,
                           jnp.zeros((1, E), jnp.float32))

    lane_e = jax.lax.broadcasted_iota(jnp.int32, (1, E), 1)
    n_s = [jnp.sum(jnp.where(lane_e == e, totals, 0.0)) for e in range(E)]
    start_s = []
    run = jnp.zeros((), jnp.float32)
    for e in range(E):
        start_s.append(run)
        run = run + jnp.floor((n_s[e] + (BLK - 1)) / BLK) * BLK

    lane8r = jax.lax.broadcasted_iota(jnp.int32, (8, 8), 1)
    cnt = jnp.zeros((8, 8), jnp.float32)
    for e in range(E):
        cnt = cnt + jnp.where(lane8r == e, n_s[e], 0.0)
    cnt_ref[...] = cnt.astype(jnp.int32)

    def pass2(i, _):
        mc = meta_ref[pl.ds(i * CB, CB), :]
        am1 = mc[:, 0:1]
        am2 = mc[:, 1:2]
        r0 = mc[:, 2:3]
        r1 = mc[:, 3:4]
        s0 = jnp.zeros_like(r0)
        s1 = jnp.zeros_like(r1)
        for e in range(E):
            st = start_s[e].astype(jnp.int32)
            s0 = jnp.where(am1 == e, st, s0)
            s1 = jnp.where(am2 == e, st, s1)
        d0 = s0 + r0
        d1 = s1 + r1
        lane8 = jax.lax.broadcasted_iota(jnp.int32, (CB, 8), 1)
        dst_ref[pl.ds(i * CB, CB), :] = (
            jnp.where(lane8 == 0, d0, 0) + jnp.where(lane8 == 1, d1, 0))
        return 0

    lax.fori_loop(0, nchunks, pass2, 0)


def _router(x, Wr, br):
    S, H = x.shape
    E = Wr.shape[1]
    body = functools.partial(_router_body, S=S, E=E, H=H)
    return pl.pallas_call(
        body,
        out_shape=[
            jax.ShapeDtypeStruct((S, 8), jnp.int32),
            jax.ShapeDtypeStruct((S, 8), jnp.float32),
            jax.ShapeDtypeStruct((8, 8), jnp.int32),
        ],
        scratch_shapes=[pltpu.VMEM((S, 8), jnp.int32)],
    )(x, Wr, br.reshape(1, E))


def _dispatch_sc(x, d0, d1, R):
    S, H = x.shape
    nchunks = S // CH
    per_w = nchunks // NW
    d0c = d0.reshape(nchunks, CH)
    d1c = d1.reshape(nchunks, CH)
    mesh = plsc.VectorSubcoreMesh(core_axis_name="c", subcore_axis_name="s")

    @functools.partial(
        pl.kernel, mesh=mesh,
        out_type=jax.ShapeDtypeStruct((R, H), jnp.float32),
        scratch_types=[
            pltpu.VMEM((CH, H), jnp.float32),
            pltpu.VMEM((CH,), jnp.int32),
            pltpu.VMEM((CH,), jnp.int32),
            pltpu.SemaphoreType.DMA,
        ],
    )
    def k(x_hbm, d0_hbm, d1_hbm, xs_hbm, xbuf, i0, i1, sem):
        wid = lax.axis_index("s") * 2 + lax.axis_index("c")
        for j in range(per_w):
            c = wid * per_w + j
            pltpu.sync_copy(x_hbm.at[pl.ds(c * CH, CH)], xbuf)
            pltpu.sync_copy(d0_hbm.at[c], i0)
            pltpu.sync_copy(d1_hbm.at[c], i1)
            pltpu.async_copy(xbuf, xs_hbm.at[i0], sem).wait()
            pltpu.async_copy(xbuf, xs_hbm.at[i1], sem).wait()

    return k(x, d0c, d1c)


def _ffn_body(g_ref, a_ref, x_ref, lng_ref, lnb_ref, w1_ref, b1_ref, w2_ref,
              b2_ref, y_ref, acc_ref, w1buf, w2buf, w1sem, w2sem,
              *, ff_t, t_max, blk_ff, n_e):
    e = pl.program_id(0)
    ff = pl.program_id(1)
    t = pl.program_id(2)
    sl = pl.ds(t * BLK, BLK)

    p = e * ff_t + ff
    npass = n_e * ff_t
    slot = lax.rem(p, 2)

    def w1_copy(pp, s):
        ee = pp // ff_t
        fz = lax.rem(pp, ff_t)
        return pltpu.make_async_copy(
            w1_ref.at[ee, :, pl.ds(fz * blk_ff, blk_ff)], w1buf.at[s],
            w1sem.at[s])

    def w2_copy(pp, s):
        ee = pp // ff_t
        fz = lax.rem(pp, ff_t)
        return pltpu.make_async_copy(
            w2_ref.at[ee, pl.ds(fz * blk_ff, blk_ff), :], w2buf.at[s],
            w2sem.at[s])

    @pl.when(t == 0)
    def _prefetch():
        @pl.when(p == 0)
        def _first():
            w1_copy(p, slot).start()
            w2_copy(p, slot).start()

        @pl.when(p + 1 < npass)
        def _next():
            w1_copy(p + 1, 1 - slot).start()
            w2_copy(p + 1, 1 - slot).start()

        w1_copy(p, slot).wait()
        w2_copy(p, slot).wait()

    @pl.when(a_ref[e * t_max + t] > 0)
    def _():
        x = x_ref[...]
        mu = jnp.mean(x, axis=-1, keepdims=True)
        var = (jnp.mean(jnp.square(x), axis=-1, keepdims=True)
               - jnp.square(mu))
        xn = (x - mu) * jax.lax.rsqrt(var + 1e-5)
        xn = (xn * lng_ref[0, 0] + lnb_ref[0, 0]).astype(jnp.bfloat16)

        h1 = jnp.dot(xn, w1buf[slot], preferred_element_type=jnp.float32)
        h1 = h1 + b1_ref[0, 0]
        h1 = 0.5 * h1 * (1.0 + jax.lax.erf(h1 * (1.0 / math.sqrt(2.0))))
        part = jnp.dot(h1.astype(jnp.bfloat16), w2buf[slot],
                       preferred_element_type=jnp.float32)

        @pl.when(ff == 0)
        def _init():
            acc_ref[sl, :] = part.astype(jnp.bfloat16)

        @pl.when(ff > 0)
        def _acc():
            acc_ref[sl, :] = (acc_ref[sl, :].astype(jnp.float32)
                              + part).astype(jnp.bfloat16)

        @pl.when(ff == ff_t - 1)
        def _flush():
            y_ref[...] = acc_ref[sl, :].astype(jnp.float32) + b2_ref[0, 0]


def _ffn_grouped(xs, g_clamp, active, ln_g, ln_b, W1, b1, W2, b2, t_max):
    R, H = xs.shape
    E, _, FF = W1.shape
    blk_ff = min(2048, FF)
    ff_t = FF // blk_ff
    W1 = W1.astype(jnp.bfloat16)
    W2 = W2.astype(jnp.bfloat16)
    body = functools.partial(_ffn_body, ff_t=ff_t, t_max=t_max,
                             blk_ff=blk_ff, n_e=E)

    def y_idx(e, ff, t, g, a):
        return (jnp.where(ff == ff_t - 1, g[e * t_max + t], g[e * t_max]), 0)

    grid_spec = pltpu.PrefetchScalarGridSpec(
        num_scalar_prefetch=2,
        grid=(E, ff_t, t_max),
        in_specs=[
            pl.BlockSpec((BLK, H), lambda e, ff, t, g, a: (g[e * t_max + t], 0)),
            pl.BlockSpec((1, 1, H), lambda e, ff, t, g, a: (e, 0, 0)),
            pl.BlockSpec((1, 1, H), lambda e, ff, t, g, a: (e, 0, 0)),
            pl.BlockSpec(memory_space=pl.ANY),
            pl.BlockSpec((1, 1, blk_ff),
                         lambda e, ff, t, g, a: (e * ff_t + ff, 0, 0)),
            pl.BlockSpec(memory_space=pl.ANY),
            pl.BlockSpec((1, 1, H), lambda e, ff, t, g, a: (e, 0, 0)),
        ],
        out_specs=pl.BlockSpec((BLK, H), y_idx),
        scratch_shapes=[
            pltpu.VMEM((t_max * BLK, H), jnp.bfloat16),
            pltpu.VMEM((2, H, blk_ff), jnp.bfloat16),
            pltpu.VMEM((2, blk_ff, H), jnp.bfloat16),
            pltpu.SemaphoreType.DMA((2,)),
            pltpu.SemaphoreType.DMA((2,)),
        ],
    )
    return pl.pallas_call(
        body,
        grid_spec=grid_spec,
        out_shape=jax.ShapeDtypeStruct((R, H), jnp.float32),
    )(g_clamp, active, xs, ln_g.reshape(E, 1, H), ln_b.reshape(E, 1, H),
      W1, b1.reshape(E * ff_t, 1, blk_ff), W2, b2.reshape(E, 1, H))


def _slot_gather_sc(y, d0, d1):
    R, H = y.shape
    S = d0.shape[0]
    nchunks = S // CH
    per_w = nchunks // NW
    d0c = d0.reshape(nchunks, CH)
    d1c = d1.reshape(nchunks, CH)
    mesh = plsc.VectorSubcoreMesh(core_axis_name="c", subcore_axis_name="s")

    @functools.partial(
        pl.kernel, mesh=mesh,
        out_type=[
            jax.ShapeDtypeStruct((S, H), jnp.float32),
            jax.ShapeDtypeStruct((S, H), jnp.float32),
        ],
        scratch_types=[
            pltpu.VMEM((CH, H), jnp.float32),
            pltpu.VMEM((CH,), jnp.int32),
            pltpu.SemaphoreType.DMA,
        ],
    )
    def k(y_hbm, d0_hbm, d1_hbm, y0_hbm, y1_hbm, buf, idx, sem):
        wid = lax.axis_index("s") * 2 + lax.axis_index("c")
        for j in range(per_w):
            c = wid * per_w + j
            pltpu.sync_copy(d0_hbm.at[c], idx)
            pltpu.async_copy(y_hbm.at[idx], buf, sem).wait()
            pltpu.sync_copy(buf, y0_hbm.at[pl.ds(c * CH, CH)])
            pltpu.sync_copy(d1_hbm.at[c], idx)
            pltpu.async_copy(y_hbm.at[idx], buf, sem).wait()
            pltpu.sync_copy(buf, y1_hbm.at[pl.ds(c * CH, CH)])

    return k(y, d0c, d1c)


def _combine_body(x_ref, y0_ref, y1_ref, w_ref, o_ref):
    w0 = w_ref[:, 0:1]
    w1 = w_ref[:, 1:2]
    o_ref[...] = x_ref[...] + w0 * y0_ref[...] + w1 * y1_ref[...]


def _combine(x, y0, y1, wgt):
    S, H = x.shape
    blk = 256
    grid = (S // blk,)
    return pl.pallas_call(
        _combine_body,
        grid=grid,
        in_specs=[
            pl.BlockSpec((blk, H), lambda s: (s, 0)),
            pl.BlockSpec((blk, H), lambda s: (s, 0)),
            pl.BlockSpec((blk, H), lambda s: (s, 0)),
            pl.BlockSpec((blk, 8), lambda s: (s, 0)),
        ],
        out_specs=pl.BlockSpec((blk, H), lambda s: (s, 0)),
        out_shape=jax.ShapeDtypeStruct((S, H), jnp.float32),
    )(x, y0, y1, wgt)


def kernel(hidden_states, Wr, br, ln_g, ln_b, W1, b1, W2, b2):
    B, S_in, H = hidden_states.shape
    S = B * S_in
    E = Wr.shape[1]
    CAP = 2
    x = hidden_states.reshape(S, H)

    dst, wgt, cnt = _router(x, Wr, br)
    d0 = dst[:, 0]
    d1 = dst[:, 1]

    t_max = S // BLK
    n = cnt[0, :E]
    tiles = (n + (BLK - 1)) // BLK
    excl = jnp.concatenate(
        [jnp.zeros((1,), jnp.int32), jnp.cumsum(tiles)[:-1].astype(jnp.int32)])
    t_iota = jnp.arange(t_max, dtype=jnp.int32)
    g_clamp = (excl[:, None]
               + jnp.minimum(t_iota[None, :],
                             jnp.maximum(tiles[:, None] - 1, 0))
               ).reshape(-1).astype(jnp.int32)
    active = (t_iota[None, :] < tiles[:, None]).reshape(-1).astype(jnp.int32)

    R = CAP * S + E * BLK
    xs = _dispatch_sc(x, d0, d1, R)
    y = _ffn_grouped(xs, g_clamp, active, ln_g, ln_b, W1, b1, W2, b2, t_max)
    y0, y1 = _slot_gather_sc(y, d0, d1)
    out = _combine(x, y0, y1, wgt)
    return out.reshape(B, S_in, H)
